# Initial kernel scaffold; baseline (speedup 1.0000x reference)
import jax
import jax.numpy as jnp
from jax.experimental import pallas as pl


def kernel(x, w1, b1, w2a, b2a, w2b, b2b, w3, b3, wo, bo):
    raise NotImplementedError("write your pallas kernel here")


# speedup vs baseline: 1.7351x; 1.7351x over previous
import jax
import jax.numpy as jnp
from jax import lax
from jax.experimental import pallas as pl
from jax.experimental.pallas import tpu as pltpu


_IN_H = _IN_W = 80
_C1, _C2 = 16, 32
_H2 = _W2 = 40
_H3 = _W3 = 20
_H4 = _W4 = 10
_WP1 = _IN_W + 4
_WP2 = _W2 + 4
_WP3 = _W3 + 4
_X1_ROWS = 84 * _WP1 + 8
_M1 = _IN_H * _WP1
_XP2_ROWS = (_H2 + 4) * _WP2 + 8
_M2 = _H2 * _WP2
_XP3_ROWS = (_H3 + 4) * _WP3 + 8
_M3 = _H3 * _WP3


def _fused_kernel(xc5_ref, w1_ref, b1_ref, w2a_ref, b2a_ref, w2b_ref,
                  b2b_ref, w3_ref, b3_ref, wo_ref, bo_ref, o_ref,
                  xp2, xp2b, xp3, y1s, y2s, y3s, flat, hb1, hb2, hb3):
    f32 = jnp.float32

    xp2[...] = jnp.zeros_like(xp2)
    xp2b[...] = jnp.zeros_like(xp2b)
    xp3[...] = jnp.zeros_like(xp3)

    acc = jnp.dot(xc5_ref[0, pl.ds(0, _M1), :], w1_ref[0],
                  preferred_element_type=f32)
    for dy in range(1, 5):
        acc = acc + jnp.dot(xc5_ref[0, pl.ds(dy * _WP1, _M1), :], w1_ref[dy],
                            preferred_element_type=f32)
    y1s[...] = jnp.maximum(acc + b1_ref[...], 0.0)

    def pool1(ho, c):
        a = y1s[pl.ds(ho * 2 * _WP1, _WP1), :]
        b = y1s[pl.ds((ho * 2 + 1) * _WP1, _WP1), :]
        hb1[...] = jnp.maximum(a, b)
        hp = jnp.maximum(hb1[pl.ds(0, _W2, stride=2), :],
                         hb1[pl.ds(1, _W2, stride=2), :])
        xp2[pl.ds((ho + 2) * _WP2 + 2, _W2), :] = hp
        return c
    lax.fori_loop(0, _H2, pool1, 0)

    acc = jnp.zeros((_M2, _C2), f32)
    for t in range(25):
        off = (t // 5) * _WP2 + (t % 5)
        acc = acc + jnp.dot(xp2[pl.ds(off, _M2), :], w2a_ref[t],
                            preferred_element_type=f32)
    y2s[...] = jnp.maximum(acc + b2a_ref[...], 0.0)

    def cp2(h, c):
        xp2b[pl.ds((h + 2) * _WP2 + 2, _W2), :] = y2s[pl.ds(h * _WP2, _W2), :]
        return c
    lax.fori_loop(0, _H2, cp2, 0)

    acc = jnp.zeros((_M2, _C2), f32)
    for t in range(25):
        off = (t // 5) * _WP2 + (t % 5)
        acc = acc + jnp.dot(xp2b[pl.ds(off, _M2), :], w2b_ref[t],
                            preferred_element_type=f32)
    y2s[...] = jnp.maximum(acc + b2b_ref[...], 0.0)

    def pool2(ho, c):
        a = y2s[pl.ds(ho * 2 * _WP2, _WP2), :]
        b = y2s[pl.ds((ho * 2 + 1) * _WP2, _WP2), :]
        hb2[...] = jnp.maximum(a, b)
        hp = jnp.maximum(hb2[pl.ds(0, _W3, stride=2), :],
                         hb2[pl.ds(1, _W3, stride=2), :])
        xp3[pl.ds((ho + 2) * _WP3 + 2, _W3), :] = hp
        return c
    lax.fori_loop(0, _H3, pool2, 0)

    acc = jnp.zeros((_M3, 1), f32)
    for t in range(25):
        off = (t // 5) * _WP3 + (t % 5)
        acc = acc + jnp.dot(xp3[pl.ds(off, _M3), :], w3_ref[t],
                            preferred_element_type=f32)
    y3s[...] = jnp.maximum(acc + b3_ref[...], 0.0)

    def pool3(ho, c):
        a = y3s[pl.ds(ho * 2 * _WP3, _WP3), :]
        b = y3s[pl.ds((ho * 2 + 1) * _WP3, _WP3), :]
        hb3[...] = jnp.maximum(a, b)
        hp = jnp.maximum(hb3[pl.ds(0, _W4, stride=2), :],
                         hb3[pl.ds(1, _W4, stride=2), :])
        flat[pl.ds(ho * _W4, _W4), :] = hp
        return c
    lax.fori_loop(0, _H4, pool3, 0)

    q = jnp.sum(flat[...] * wo_ref[...], axis=0, keepdims=True) + bo_ref[...]
    o_ref[...] = q.reshape(1, 1, -1).astype(o_ref.dtype)


def kernel(x, w1, b1, w2a, b2a, w2b, b2b, w3, b3, wo, bo):
    B = x.shape[0]
    A = wo.shape[1]

    xp = jnp.pad(x[:, 0], ((0, 0), (2, 2), (2, 2))).reshape(B, 84 * _WP1)
    xf = jnp.pad(xp, ((0, 0), (0, _X1_ROWS - 84 * _WP1 + 4)))
    xc5 = jnp.stack([xf[:, dx:dx + _X1_ROWS] for dx in range(5)], axis=-1)

    w1m = w1.reshape(5, 5, _C1)
    w2am = w2a.reshape(25, _C1, _C2)
    w2bm = w2b.reshape(25, _C2, _C2)
    w3m = w3.reshape(25, _C2, 1)
    b1m = b1.reshape(1, _C1)
    b2am = b2a.reshape(1, _C2)
    b2bm = b2b.reshape(1, _C2)
    b3m = b3.reshape(1, 1)
    bom = bo.reshape(1, A)

    def full(shape):
        return pl.BlockSpec(shape, lambda b, _s=shape: (0,) * len(_s))

    out = pl.pallas_call(
        _fused_kernel,
        out_shape=jax.ShapeDtypeStruct((B, 1, A), jnp.float32),
        grid=(B,),
        in_specs=[
            pl.BlockSpec((1, _X1_ROWS, 5), lambda b: (b, 0, 0)),
            full((5, 5, _C1)), full((1, _C1)),
            full((25, _C1, _C2)), full((1, _C2)),
            full((25, _C2, _C2)), full((1, _C2)),
            full((25, _C2, 1)), full((1, 1)),
            full((_H4 * _W4, A)), full((1, A)),
        ],
        out_specs=pl.BlockSpec((1, 1, A), lambda b: (b, 0, 0)),
        scratch_shapes=[
            pltpu.VMEM((_XP2_ROWS, _C1), jnp.float32),
            pltpu.VMEM((_XP2_ROWS, _C2), jnp.float32),
            pltpu.VMEM((_XP3_ROWS, _C2), jnp.float32),
            pltpu.VMEM((_M1, _C1), jnp.float32),
            pltpu.VMEM((_M2, _C2), jnp.float32),
            pltpu.VMEM((_M3, 1), jnp.float32),
            pltpu.VMEM((_H4 * _W4, 1), jnp.float32),
            pltpu.VMEM((_WP1, _C1), jnp.float32),
            pltpu.VMEM((_WP2, _C2), jnp.float32),
            pltpu.VMEM((_WP3, 1), jnp.float32),
        ],
        compiler_params=pltpu.CompilerParams(
            dimension_semantics=("parallel",),
            vmem_limit_bytes=64 * 1024 * 1024),
    )(xc5, w1m, b1m, w2am, b2am, w2bm, b2bm, w3m, b3m, wo, bom)
    return out.reshape(B, A)


# speedup vs baseline: 1.8848x; 1.0863x over previous
import jax
import jax.numpy as jnp
from jax.experimental import pallas as pl
from jax.experimental.pallas import tpu as pltpu


_IN_H = _IN_W = 80
_C1, _C2 = 16, 32
_H2 = _W2 = 40
_H3 = _W3 = 20
_H4 = _W4 = 10
_WP1 = _IN_W + 4
_WP2 = _W2 + 4
_WP3 = _W3 + 4
_X1_ROWS = 84 * _WP1 + 8
_M1 = _IN_H * _WP1
_XP2_ROWS = (_H2 + 4) * _WP2 + 8
_M2 = _H2 * _WP2
_XP3_ROWS = (_H3 + 4) * _WP3 + 8
_M3 = _H3 * _WP3


def _fused_kernel(xc5_ref, w1_ref, b1_ref, w2a_ref, b2a_ref, w2b_ref,
                  b2b_ref, w3_ref, b3_ref, wo_ref, bo_ref, o_ref,
                  xp2, xp2b, xp3, y1s, y2s, y3s, flat, hb1, hb2, hb3):
    f32 = jnp.float32
    bf16 = jnp.bfloat16

    xp2[...] = jnp.zeros_like(xp2)
    xp2b[...] = jnp.zeros_like(xp2b)
    xp3[...] = jnp.zeros_like(xp3)

    acc = jnp.dot(xc5_ref[0, pl.ds(0, _M1), :], w1_ref[0],
                  preferred_element_type=f32)
    for dy in range(1, 5):
        acc = acc + jnp.dot(xc5_ref[0, pl.ds(dy * _WP1, _M1), :], w1_ref[dy],
                            preferred_element_type=f32)
    y1s[...] = jnp.maximum(acc + b1_ref[...], 0.0)

    for ho in range(_H2):
        a = y1s[pl.ds(ho * 2 * _WP1, _WP1), :]
        b = y1s[pl.ds((ho * 2 + 1) * _WP1, _WP1), :]
        hb1[...] = jnp.maximum(a, b)
        hp = jnp.maximum(hb1[pl.ds(0, _W2, stride=2), :],
                         hb1[pl.ds(1, _W2, stride=2), :])
        xp2[pl.ds((ho + 2) * _WP2 + 2, _W2), :] = hp.astype(bf16)

    acc = jnp.zeros((_M2, _C2), f32)
    for t in range(25):
        off = (t // 5) * _WP2 + (t % 5)
        acc = acc + jnp.dot(xp2[pl.ds(off, _M2), :], w2a_ref[t],
                            preferred_element_type=f32)
    y2s[...] = jnp.maximum(acc + b2a_ref[...], 0.0)

    for h in range(_H2):
        xp2b[pl.ds((h + 2) * _WP2 + 2, _W2), :] = (
            y2s[pl.ds(h * _WP2, _W2), :].astype(bf16))

    acc = jnp.zeros((_M2, _C2), f32)
    for t in range(25):
        off = (t // 5) * _WP2 + (t % 5)
        acc = acc + jnp.dot(xp2b[pl.ds(off, _M2), :], w2b_ref[t],
                            preferred_element_type=f32)
    y2s[...] = jnp.maximum(acc + b2b_ref[...], 0.0)

    for ho in range(_H3):
        a = y2s[pl.ds(ho * 2 * _WP2, _WP2), :]
        b = y2s[pl.ds((ho * 2 + 1) * _WP2, _WP2), :]
        hb2[...] = jnp.maximum(a, b)
        hp = jnp.maximum(hb2[pl.ds(0, _W3, stride=2), :],
                         hb2[pl.ds(1, _W3, stride=2), :])
        xp3[pl.ds((ho + 2) * _WP3 + 2, _W3), :] = hp.astype(bf16)

    acc = jnp.zeros((_M3, 1), f32)
    for t in range(25):
        off = (t // 5) * _WP3 + (t % 5)
        acc = acc + jnp.dot(xp3[pl.ds(off, _M3), :], w3_ref[t],
                            preferred_element_type=f32)
    y3s[...] = jnp.maximum(acc + b3_ref[...], 0.0)

    for ho in range(_H4):
        a = y3s[pl.ds(ho * 2 * _WP3, _WP3), :]
        b = y3s[pl.ds((ho * 2 + 1) * _WP3, _WP3), :]
        hb3[...] = jnp.maximum(a, b)
        hp = jnp.maximum(hb3[pl.ds(0, _W4, stride=2), :],
                         hb3[pl.ds(1, _W4, stride=2), :])
        flat[pl.ds(ho * _W4, _W4), :] = hp

    q = jnp.sum(flat[...] * wo_ref[...], axis=0, keepdims=True) + bo_ref[...]
    o_ref[...] = q.reshape(1, 1, -1).astype(o_ref.dtype)


def kernel(x, w1, b1, w2a, b2a, w2b, b2b, w3, b3, wo, bo):
    B = x.shape[0]
    A = wo.shape[1]
    bf16 = jnp.bfloat16

    xp = jnp.pad(x[:, 0], ((0, 0), (2, 2), (2, 2))).reshape(B, 84 * _WP1)
    xf = jnp.pad(xp, ((0, 0), (0, _X1_ROWS - 84 * _WP1 + 4)))
    xc5 = jnp.stack(
        [xf[:, dx:dx + _X1_ROWS] for dx in range(5)], axis=-1).astype(bf16)

    w1m = w1.reshape(5, 5, _C1).astype(bf16)
    w2am = w2a.reshape(25, _C1, _C2).astype(bf16)
    w2bm = w2b.reshape(25, _C2, _C2).astype(bf16)
    w3m = w3.reshape(25, _C2, 1).astype(bf16)
    b1m = b1.reshape(1, _C1)
    b2am = b2a.reshape(1, _C2)
    b2bm = b2b.reshape(1, _C2)
    b3m = b3.reshape(1, 1)
    bom = bo.reshape(1, A)

    def full(shape):
        return pl.BlockSpec(shape, lambda b, _s=shape: (0,) * len(_s))

    out = pl.pallas_call(
        _fused_kernel,
        out_shape=jax.ShapeDtypeStruct((B, 1, A), jnp.float32),
        grid=(B,),
        in_specs=[
            pl.BlockSpec((1, _X1_ROWS, 5), lambda b: (b, 0, 0)),
            full((5, 5, _C1)), full((1, _C1)),
            full((25, _C1, _C2)), full((1, _C2)),
            full((25, _C2, _C2)), full((1, _C2)),
            full((25, _C2, 1)), full((1, 1)),
            full((_H4 * _W4, A)), full((1, A)),
        ],
        out_specs=pl.BlockSpec((1, 1, A), lambda b: (b, 0, 0)),
        scratch_shapes=[
            pltpu.VMEM((_XP2_ROWS, _C1), bf16),
            pltpu.VMEM((_XP2_ROWS, _C2), bf16),
            pltpu.VMEM((_XP3_ROWS, _C2), bf16),
            pltpu.VMEM((_M1, _C1), jnp.float32),
            pltpu.VMEM((_M2, _C2), jnp.float32),
            pltpu.VMEM((_M3, 1), jnp.float32),
            pltpu.VMEM((_H4 * _W4, 1), jnp.float32),
            pltpu.VMEM((_WP1, _C1), jnp.float32),
            pltpu.VMEM((_WP2, _C2), jnp.float32),
            pltpu.VMEM((_WP3, 1), jnp.float32),
        ],
        compiler_params=pltpu.CompilerParams(
            dimension_semantics=("parallel",),
            vmem_limit_bytes=64 * 1024 * 1024),
    )(xc5, w1m, b1m, w2am, b2am, w2bm, b2bm, w3m, b3m, wo, bom)
    return out.reshape(B, A)
